# baseline (speedup 1.0000x reference)
import jax
import jax.numpy as jnp
from jax import lax
from jax.experimental import pallas as pl
from jax.experimental.pallas import tpu as pltpu
from jax.experimental.pallas import tpu_sc as plsc

MAXLEN = 200
EMBED = 32
BATCH = 4096

NC, NS = 2, 16
NW = NC * NS
BC = 256
CPP = BATCH // BC
UNITS = MAXLEN * CPP
UPW = UNITS // NW
EB = EMBED // 8
BB = BC // 128
GPU_ = BC // 128


def _emb_body(x4, table, pos, out, idx_all, rows, slabs, pos_u,
              semg, sems):
    wid = lax.axis_index("s") * NC + lax.axis_index("c")
    k0 = wid * UPW
    pbase = k0 // CPP
    for i in range(8):
        pld = jnp.minimum(pbase + i, MAXLEN - 1)
        pltpu.sync_copy(x4.at[pld // 8, :, pld % 8], idx_all.at[i])

    iota = lax.iota(jnp.int32, 16)
    e1_lo = lax.shift_right_logical(iota, 3)
    e1_hi = e1_lo + 2
    e0_idx = lax.bitwise_and(iota, 7)
    ones = jnp.full((16,), 1, jnp.int32)

    def fire_gathers(k, par):
        u = k0 + k
        pi = u // CPP - pbase
        c = u % CPP
        for j in range(GPU_):
            pltpu.async_copy(
                table.at[idx_all.at[pi, c * BB + j]],
                rows[par].at[pl.ds(j * 128, 128)],
                semg[par],
            )

    def drain_gathers(par):
        for j in range(GPU_):
            pltpu.make_async_copy(
                table.at[idx_all.at[0, 0]],
                rows[par].at[pl.ds(j * 128, 128)],
                semg[par],
            ).wait()

    def drain_scatter(par):
        pltpu.make_async_copy(
            slabs[par], out.at[0, :, pl.ds(0, BB)], sems[par]
        ).wait()

    def process_unit(k, par):
        u = k0 + k
        p = u // CPP
        c = u % CPP
        pltpu.sync_copy(pos.at[p], pos_u)
        pos_lo = pos_u[pl.ds(0, 16)]
        pos_hi = pos_u[pl.ds(16, 16)]
        rows_v = rows[par]
        slab_v = slabs[par]
        for b1l in range(BB):
            b1_idx = jnp.full((16,), b1l, jnp.int32)

            @plsc.parallel_loop(
                0, 128, 1, unroll=8, carry=jnp.full((16,), 0, jnp.int32)
            )
            def t_body(i, b0vec, b1l=b1l, b1_idx=b1_idx):
                t = b1l * 128 + i
                lo0 = rows_v[t, pl.ds(0, 16)] + pos_lo
                hi0 = rows_v[t, pl.ds(16, 16)] + pos_hi
                plsc.store_scatter(slab_v, [e1_lo, b1_idx, e0_idx, b0vec], lo0)
                plsc.store_scatter(slab_v, [e1_hi, b1_idx, e0_idx, b0vec], hi0)
                return b0vec + ones

        pltpu.async_copy(slab_v, out.at[p, :, pl.ds(c * BB, BB)], sems[par])

    fire_gathers(0, 0)

    def pair_body(g, carry):
        for par in range(2):
            k = g * 2 + par

            @pl.when(k + 1 < UPW)
            def _fire():
                fire_gathers(k + 1, 1 - par)

            drain_gathers(par)

            @pl.when(k >= 2)
            def _drain_s():
                drain_scatter(par)

            process_unit(k, par)
        return carry

    lax.fori_loop(0, UPW // 2, pair_body, 0)
    drain_scatter(0)
    drain_scatter(1)


@jax.jit
def kernel(x, token_table, pos_table):
    x4 = (
        x.astype(jnp.int32)
        .T.reshape(MAXLEN // 8, 8, BATCH // 128, 128)
        .transpose(0, 2, 1, 3)
    )
    tpad = jnp.pad(token_table, ((0, 0), (0, 128 - EMBED)))
    mesh = plsc.VectorSubcoreMesh(core_axis_name="c", subcore_axis_name="s")
    out5 = pl.kernel(
        _emb_body,
        out_type=jax.ShapeDtypeStruct((MAXLEN, EB, BATCH // 128, 8, 128), jnp.float32),
        mesh=mesh,
        compiler_params=pltpu.CompilerParams(
            use_tc_tiling_on_sc=True, needs_layout_passes=False
        ),
        scratch_types=[
            pltpu.VMEM((8, EMBED, 128), jnp.int32),
            [pltpu.VMEM((BC, 128), jnp.float32)] * 2,
            [pltpu.VMEM((EB, BB, 8, 128), jnp.float32)] * 2,
            pltpu.VMEM((EMBED,), jnp.float32),
            [pltpu.SemaphoreType.DMA] * 2,
            [pltpu.SemaphoreType.DMA] * 2,
        ],
    )(x4, tpad, pos_table)
    return out5.transpose(2, 4, 0, 1, 3).reshape(BATCH, MAXLEN, EMBED)


# speedup vs baseline: 1.4170x; 1.4170x over previous
import jax
import jax.numpy as jnp
from jax import lax
from jax.experimental import pallas as pl
from jax.experimental.pallas import tpu as pltpu
from jax.experimental.pallas import tpu_sc as plsc

MAXLEN = 200
EMBED = 32
BATCH = 4096

NC, NS = 2, 16
NW = NC * NS
BC = 512
CPP = BATCH // BC
UNITS = MAXLEN * CPP
UPW = UNITS // NW
EB = EMBED // 8
BB = BC // 128
GPU_ = BC // 128


def _emb_body(x4, table, pos, out, idx_all, rows, slabs, pos_v,
              semg, sems):
    wid = lax.axis_index("s") * NC + lax.axis_index("c")
    k0 = wid * UPW
    pbase = k0 // CPP
    pltpu.sync_copy(pos, pos_v)
    for i in range(8):
        pld = jnp.minimum(pbase + i, MAXLEN - 1)
        pltpu.sync_copy(x4.at[pld // 8, :, pld % 8], idx_all.at[i])

    iota = lax.iota(jnp.int32, 16)
    e1_lo = lax.shift_right_logical(iota, 3)
    e1_hi = e1_lo + 2
    e0_idx = lax.bitwise_and(iota, 7)
    ones = jnp.full((16,), 1, jnp.int32)

    def fire_gathers(k, par):
        u = k0 + k
        pi = u // CPP - pbase
        c = u % CPP
        for j in range(GPU_ * 2):
            pltpu.async_copy(
                table.at[idx_all.at[pi, c * BB + j // 2, pl.ds((j % 2) * 64, 64)]],
                rows[par].at[pl.ds(j * 64, 64)],
                semg[par],
            )

    def drain_gathers(par):
        for j in range(GPU_):
            pltpu.make_async_copy(
                table.at[idx_all.at[0, 0]],
                rows[par].at[pl.ds(j * 128, 128)],
                semg[par],
            ).wait()

    def drain_scatter(par):
        pltpu.make_async_copy(
            slabs[par], out.at[0, :, pl.ds(0, BB)], sems[par]
        ).wait()

    def process_unit(k, par):
        u = k0 + k
        p = u // CPP
        c = u % CPP
        pos_lo = pos_v[p, pl.ds(0, 16)]
        pos_hi = pos_v[p, pl.ds(16, 16)]
        rows_v = rows[par]
        slab_v = slabs[par]

        @plsc.parallel_loop(0, BC, 1, unroll=8)
        def pos_body(t):
            rows_v[t, pl.ds(0, 16)] = rows_v[t, pl.ds(0, 16)] + pos_lo
            rows_v[t, pl.ds(16, 16)] = rows_v[t, pl.ds(16, 16)] + pos_hi

        @plsc.parallel_loop(0, BC // 16, 1)
        def blk_body(g):
            base = g * 16
            row_idx = iota + base
            b1v = jnp.full((16,), g // 8, jnp.int32)
            b0v = iota + (g % 8) * 16
            for h in range(2):
                for r in range(16):
                    col = lax.bitwise_and(iota + r, 15)
                    if h:
                        col = lax.bitwise_or(col, 16)
                    val = plsc.load_gather(rows_v, [row_idx, col])
                    e1v = lax.shift_right_logical(col, 3)
                    e0v = lax.bitwise_and(col, 7)
                    plsc.store_scatter(slab_v, [e1v, b1v, e0v, b0v], val)
        pltpu.async_copy(slab_v, out.at[p, :, pl.ds(c * BB, BB)], sems[par])

    fire_gathers(0, 0)

    def pair_body(g, carry):
        for par in range(2):
            k = g * 2 + par

            @pl.when(k + 1 < UPW)
            def _fire():
                fire_gathers(k + 1, 1 - par)

            drain_gathers(par)

            @pl.when(k >= 2)
            def _drain_s():
                drain_scatter(par)

            process_unit(k, par)
        return carry

    lax.fori_loop(0, UPW // 2, pair_body, 0)
    drain_scatter(0)
    drain_scatter(1)


@jax.jit
def kernel(x, token_table, pos_table):
    x4 = (
        x.astype(jnp.int32)
        .T.reshape(MAXLEN // 8, 8, BATCH // 128, 128)
        .transpose(0, 2, 1, 3)
    )
    mesh = plsc.VectorSubcoreMesh(core_axis_name="c", subcore_axis_name="s")
    out5 = pl.kernel(
        _emb_body,
        out_type=jax.ShapeDtypeStruct((MAXLEN, EB, BATCH // 128, 8, 128), jnp.float32),
        mesh=mesh,
        compiler_params=pltpu.CompilerParams(
            use_tc_tiling_on_sc=False, needs_layout_passes=False
        ),
        scratch_types=[
            pltpu.VMEM((8, EMBED, 128), jnp.int32),
            [pltpu.VMEM((BC, EMBED), jnp.float32)] * 2,
            [pltpu.VMEM((EB, BB, 8, 128), jnp.float32)] * 2,
            pltpu.VMEM((MAXLEN, EMBED), jnp.float32),
            [pltpu.SemaphoreType.DMA] * 2,
            [pltpu.SemaphoreType.DMA] * 2,
        ],
    )(x4, token_table, pos_table)
    return out5.transpose(2, 4, 0, 1, 3).reshape(BATCH, MAXLEN, EMBED)


# speedup vs baseline: 2.8816x; 2.0336x over previous
import jax
import jax.numpy as jnp
from jax import lax
from jax.experimental import pallas as pl
from jax.experimental.pallas import tpu as pltpu
from jax.experimental.pallas import tpu_sc as plsc

MAXLEN = 200
EMBED = 32
BATCH = 4096

NC, NS = 2, 16
NW = NC * NS
BC = 512
CPP = BATCH // BC
UNITS = MAXLEN * CPP
UPW = UNITS // NW
EB = EMBED // 8
BB = BC // 128
GPU_ = BC // 128


def _emb_body(x4, table, pos, out, idx_all, rows, slabs, pos_v,
              semg, sems):
    wid = lax.axis_index("s") * NC + lax.axis_index("c")
    k0 = wid * UPW
    pbase = k0 // CPP
    pltpu.sync_copy(pos, pos_v)
    for i in range(8):
        pld = jnp.minimum(pbase + i, MAXLEN - 1)
        pltpu.sync_copy(x4.at[pld // 8, :, pld % 8], idx_all.at[i])

    iota = lax.iota(jnp.int32, 16)
    e1_lo = lax.shift_right_logical(iota, 3)
    e1_hi = e1_lo + 2
    e0_idx = lax.bitwise_and(iota, 7)
    ones = jnp.full((16,), 1, jnp.int32)

    def fire_gathers(k, par):
        u = k0 + k
        pi = u // CPP - pbase
        c = u % CPP
        for j in range(GPU_ * 2):
            pltpu.async_copy(
                table.at[idx_all.at[pi, c * BB + j // 2, pl.ds((j % 2) * 64, 64)]],
                rows[par].at[pl.ds(j * 64, 64)],
                semg[par],
            )

    def drain_gathers(par):
        for j in range(GPU_):
            pltpu.make_async_copy(
                table.at[idx_all.at[0, 0]],
                rows[par].at[pl.ds(j * 128, 128)],
                semg[par],
            ).wait()

    def drain_scatter(par):
        pltpu.make_async_copy(
            slabs[par], out.at[0, :, pl.ds(0, BB)], sems[par]
        ).wait()

    def process_unit(k, par):
        u = k0 + k
        p = u // CPP
        c = u % CPP
        pos_lo = pos_v[p, pl.ds(0, 16)]
        pos_hi = pos_v[p, pl.ds(16, 16)]
        rows_v = rows[par]
        slab_v = slabs[par]

        @plsc.parallel_loop(0, BC, 1, unroll=8)
        def pos_body(t):
            rows_v[t, pl.ds(0, 16)] = rows_v[t, pl.ds(0, 16)] + pos_lo
            rows_v[t, pl.ds(16, 16)] = rows_v[t, pl.ds(16, 16)] + pos_hi

        @plsc.parallel_loop(0, BC // 16, 1)
        def blk_body(g):
            base = g * 16
            row_idx = iota + base
            b1v = jnp.full((16,), g // 8, jnp.int32)
            b0v = iota + (g % 8) * 16
            for h in range(2):
                for r in range(16):
                    col = lax.bitwise_and(iota + r, 15)
                    if h:
                        col = lax.bitwise_or(col, 16)
                    val = plsc.load_gather(rows_v, [row_idx, col])
                    e1v = lax.shift_right_logical(col, 3)
                    e0v = lax.bitwise_and(col, 7)
                    plsc.store_scatter(slab_v, [e1v, b1v, e0v, b0v], val)
        pltpu.async_copy(slab_v, out.at[p, :, pl.ds(c * BB, BB)], sems[par])

    fire_gathers(0, 0)

    def pair_body(g, carry):
        for par in range(2):
            k = g * 2 + par

            @pl.when(k + 1 < UPW)
            def _fire():
                fire_gathers(k + 1, 1 - par)

            drain_gathers(par)

            @pl.when(k >= 2)
            def _drain_s():
                drain_scatter(par)

            process_unit(k, par)
        return carry

    lax.fori_loop(0, UPW // 2, pair_body, 0)
    drain_scatter(0)
    drain_scatter(1)


VOCAB = 1000000
WCH = 512
NFULL = 1953
TAILW = VOCAB - NFULL * WCH


def _reformat_body(tT, tail1d, outf, ins, outs, tail_v, semi, semo):
    wid = lax.axis_index("s") * NC + lax.axis_index("c")
    iota = lax.iota(jnp.int32, 16)

    def chunk_id(k):
        return wid + NW * k

    def fire_in(k, par):
        ch = chunk_id(k)
        pltpu.async_copy(tT.at[:, pl.ds(ch * WCH, WCH)], ins[par], semi[par])

    def drain_in(par):
        pltpu.make_async_copy(tT.at[:, pl.ds(0, WCH)], ins[par], semi[par]).wait()

    def drain_out(par):
        pltpu.make_async_copy(
            outs[par], outf.at[pl.ds(0, WCH * EMBED)], semo[par]
        ).wait()

    def process(k, par, width):
        ch = chunk_id(k)
        in_v = ins[par]
        out_v = outs[par]

        @plsc.parallel_loop(0, width // 16, 1)
        def blk(g2):
            tv = lax.shift_left(g2 * 16 + iota, 5)
            for h in range(2):
                for r in range(16):
                    col = lax.bitwise_and(iota + r, 15)
                    if h:
                        col = lax.bitwise_or(col, 16)
                    val = plsc.load_gather(in_v, [col, g2 * 16 + iota])
                    plsc.store_scatter(out_v, [tv + col], val)

        nel = width * EMBED
        pltpu.async_copy(
            out_v.at[pl.ds(0, nel)],
            outf.at[pl.ds(ch * (WCH * EMBED), nel)],
            semo[par],
        )

    fire_in(0, 0)

    def pair_body(g, carry):
        for par in range(2):
            k = g * 2 + par

            @pl.when(chunk_id(k + 1) < NFULL)
            def _fire():
                fire_in(k + 1, 1 - par)

            @pl.when(chunk_id(k) < NFULL)
            def _work():
                drain_in(par)

                @pl.when(k >= 2)
                def _dr():
                    drain_out(par)

                process(k, par, WCH)

        return carry

    lax.fori_loop(0, 31, pair_body, 0)

    @pl.when(wid == 0)
    def _tail():
        pltpu.sync_copy(tail1d, tail_v)
        pltpu.sync_copy(tail_v, outf.at[pl.ds(NFULL * WCH * EMBED, TAILW * EMBED)])

    drain_out(0)
    drain_out(1)


def _reformat_table(token_table):
    mesh = plsc.VectorSubcoreMesh(core_axis_name="c", subcore_axis_name="s")
    flat = pl.kernel(
        _reformat_body,
        out_type=jax.ShapeDtypeStruct((VOCAB * EMBED,), jnp.float32),
        mesh=mesh,
        compiler_params=pltpu.CompilerParams(
            use_tc_tiling_on_sc=True, needs_layout_passes=False
        ),
        scratch_types=[
            [pltpu.VMEM((EMBED, WCH), jnp.float32)] * 2,
            [pltpu.VMEM((WCH * EMBED,), jnp.float32)] * 2,
            pltpu.VMEM((TAILW * EMBED,), jnp.float32),
            [pltpu.SemaphoreType.DMA] * 2,
            [pltpu.SemaphoreType.DMA] * 2,
        ],
    )(token_table.T, token_table[NFULL * WCH :].reshape(-1))
    return flat.reshape(VOCAB, EMBED)


@jax.jit
def kernel(x, token_table, pos_table):
    x4 = (
        x.astype(jnp.int32)
        .T.reshape(MAXLEN // 8, 8, BATCH // 128, 128)
        .transpose(0, 2, 1, 3)
    )
    table_flat = _reformat_table(token_table)
    mesh = plsc.VectorSubcoreMesh(core_axis_name="c", subcore_axis_name="s")
    out5 = pl.kernel(
        _emb_body,
        out_type=jax.ShapeDtypeStruct((MAXLEN, EB, BATCH // 128, 8, 128), jnp.float32),
        mesh=mesh,
        compiler_params=pltpu.CompilerParams(
            use_tc_tiling_on_sc=False, needs_layout_passes=False
        ),
        scratch_types=[
            pltpu.VMEM((8, EMBED, 128), jnp.int32),
            [pltpu.VMEM((BC, EMBED), jnp.float32)] * 2,
            [pltpu.VMEM((EB, BB, 8, 128), jnp.float32)] * 2,
            pltpu.VMEM((MAXLEN, EMBED), jnp.float32),
            [pltpu.SemaphoreType.DMA] * 2,
            [pltpu.SemaphoreType.DMA] * 2,
        ],
    )(x4, table_flat, pos_table)
    return out5.transpose(2, 4, 0, 1, 3).reshape(BATCH, MAXLEN, EMBED)


# speedup vs baseline: 3.7142x; 1.2889x over previous
import jax
import jax.numpy as jnp
from jax import lax
from jax.experimental import pallas as pl
from jax.experimental.pallas import tpu as pltpu
from jax.experimental.pallas import tpu_sc as plsc

MAXLEN = 200
EMBED = 32
BATCH = 4096

NC, NS = 2, 16
NW = NC * NS
BC = 512
CPP = BATCH // BC
UNITS = MAXLEN * CPP
UPW = UNITS // NW
EB = EMBED // 8
BB = BC // 128
GPU_ = BC // 128


def _emb_body(x4, table, pos, out, idx_all, rows, slabs, pos_v,
              semg, sems):
    wid = lax.axis_index("s") * NC + lax.axis_index("c")
    k0 = wid * UPW
    pbase = k0 // CPP
    pltpu.sync_copy(pos, pos_v)
    for i in range(8):
        pld = jnp.minimum(pbase + i, MAXLEN - 1)
        pltpu.sync_copy(x4.at[pld // 8, :, pld % 8], idx_all.at[i])

    iota = lax.iota(jnp.int32, 16)
    e1_lo = lax.shift_right_logical(iota, 3)
    e1_hi = e1_lo + 2
    e0_idx = lax.bitwise_and(iota, 7)
    ones = jnp.full((16,), 1, jnp.int32)

    def fire_gathers(k, par):
        u = k0 + k
        pi = u // CPP - pbase
        c = u % CPP
        for j in range(GPU_ * 2):
            pltpu.async_copy(
                table.at[idx_all.at[pi, c * BB + j // 2, pl.ds((j % 2) * 64, 64)]],
                rows[par].at[pl.ds(j * 64, 64)],
                semg[par],
            )

    def drain_gathers(par):
        for j in range(GPU_):
            pltpu.make_async_copy(
                table.at[idx_all.at[0, 0]],
                rows[par].at[pl.ds(j * 128, 128)],
                semg[par],
            ).wait()

    def drain_scatter(par):
        pltpu.make_async_copy(
            slabs[par], out.at[0, :, pl.ds(0, BB)], sems[par]
        ).wait()

    def process_unit(k, par):
        u = k0 + k
        p = u // CPP
        c = u % CPP
        pos_lo = pos_v[p, pl.ds(0, 16)]
        pos_hi = pos_v[p, pl.ds(16, 16)]
        rows_v = rows[par]
        slab_v = slabs[par]

        @plsc.parallel_loop(0, BC, 1, unroll=8)
        def pos_body(t):
            rows_v[t, pl.ds(0, 16)] = rows_v[t, pl.ds(0, 16)] + pos_lo
            rows_v[t, pl.ds(16, 16)] = rows_v[t, pl.ds(16, 16)] + pos_hi

        @plsc.parallel_loop(0, BC // 16, 1, unroll=2)
        def blk_body(g):
            base = g * 16
            row_idx = iota + base
            b1v = jnp.full((16,), g // 8, jnp.int32)
            b0v = iota + (g % 8) * 16
            for h in range(2):
                for r in range(16):
                    col = lax.bitwise_and(iota + r, 15)
                    if h:
                        col = lax.bitwise_or(col, 16)
                    val = plsc.load_gather(rows_v, [row_idx, col])
                    e1v = lax.shift_right_logical(col, 3)
                    e0v = lax.bitwise_and(col, 7)
                    plsc.store_scatter(slab_v, [e1v, b1v, e0v, b0v], val)
        pltpu.async_copy(slab_v, out.at[p, :, pl.ds(c * BB, BB)], sems[par])

    fire_gathers(0, 0)

    def pair_body(g, carry):
        for par in range(2):
            k = g * 2 + par

            @pl.when(k + 1 < UPW)
            def _fire():
                fire_gathers(k + 1, 1 - par)

            drain_gathers(par)

            @pl.when(k >= 2)
            def _drain_s():
                drain_scatter(par)

            process_unit(k, par)
        return carry

    lax.fori_loop(0, UPW // 2, pair_body, 0)
    drain_scatter(0)
    drain_scatter(1)


VOCAB = 1000000
WCH = 512
NFULL = 1953
TAILW = VOCAB - NFULL * WCH


def _reformat_body(tT, tail1d, outf, ins, outs, tail_v, semi, semo):
    wid = lax.axis_index("s") * NC + lax.axis_index("c")
    iota = lax.iota(jnp.int32, 16)

    def chunk_id(k):
        return wid + NW * k

    def fire_in(k, par):
        ch = chunk_id(k)
        pltpu.async_copy(tT.at[:, pl.ds(ch * WCH, WCH)], ins[par], semi[par])

    def drain_in(par):
        pltpu.make_async_copy(tT.at[:, pl.ds(0, WCH)], ins[par], semi[par]).wait()

    def drain_out(par):
        pltpu.make_async_copy(
            outs[par], outf.at[pl.ds(0, WCH * EMBED)], semo[par]
        ).wait()

    def process(k, par, width):
        ch = chunk_id(k)
        in_v = ins[par]
        out_v = outs[par]

        @plsc.parallel_loop(0, width // 16, 1, unroll=2)
        def blk(g2):
            tv = lax.shift_left(g2 * 16 + iota, 5)
            for h in range(2):
                for r in range(16):
                    col = lax.bitwise_and(iota + r, 15)
                    if h:
                        col = lax.bitwise_or(col, 16)
                    val = plsc.load_gather(in_v, [col, g2 * 16 + iota])
                    plsc.store_scatter(out_v, [tv + col], val)

        nel = width * EMBED
        pltpu.async_copy(
            out_v.at[pl.ds(0, nel)],
            outf.at[pl.ds(ch * (WCH * EMBED), nel)],
            semo[par],
        )

    fire_in(0, 0)

    def pair_body(g, carry):
        for par in range(2):
            k = g * 2 + par

            @pl.when(chunk_id(k + 1) < NFULL)
            def _fire():
                fire_in(k + 1, 1 - par)

            @pl.when(chunk_id(k) < NFULL)
            def _work():
                drain_in(par)

                @pl.when(k >= 2)
                def _dr():
                    drain_out(par)

                process(k, par, WCH)

        return carry

    lax.fori_loop(0, 31, pair_body, 0)

    @pl.when(wid == 0)
    def _tail():
        pltpu.sync_copy(tail1d, tail_v)
        pltpu.sync_copy(tail_v, outf.at[pl.ds(NFULL * WCH * EMBED, TAILW * EMBED)])

    drain_out(0)
    drain_out(1)


def _reformat_table(token_table):
    mesh = plsc.VectorSubcoreMesh(core_axis_name="c", subcore_axis_name="s")
    flat = pl.kernel(
        _reformat_body,
        out_type=jax.ShapeDtypeStruct((VOCAB * EMBED,), jnp.float32),
        mesh=mesh,
        compiler_params=pltpu.CompilerParams(
            use_tc_tiling_on_sc=True, needs_layout_passes=False
        ),
        scratch_types=[
            [pltpu.VMEM((EMBED, WCH), jnp.float32)] * 2,
            [pltpu.VMEM((WCH * EMBED,), jnp.float32)] * 2,
            pltpu.VMEM((TAILW * EMBED,), jnp.float32),
            [pltpu.SemaphoreType.DMA] * 2,
            [pltpu.SemaphoreType.DMA] * 2,
        ],
    )(token_table.T, token_table[NFULL * WCH :].reshape(-1))
    return flat.reshape(VOCAB, EMBED)


@jax.jit
def kernel(x, token_table, pos_table):
    x4 = (
        x.astype(jnp.int32)
        .T.reshape(MAXLEN // 8, 8, BATCH // 128, 128)
        .transpose(0, 2, 1, 3)
    )
    table_flat = _reformat_table(token_table)
    mesh = plsc.VectorSubcoreMesh(core_axis_name="c", subcore_axis_name="s")
    out5 = pl.kernel(
        _emb_body,
        out_type=jax.ShapeDtypeStruct((MAXLEN, EB, BATCH // 128, 8, 128), jnp.float32),
        mesh=mesh,
        compiler_params=pltpu.CompilerParams(
            use_tc_tiling_on_sc=False, needs_layout_passes=False
        ),
        scratch_types=[
            pltpu.VMEM((8, EMBED, 128), jnp.int32),
            [pltpu.VMEM((BC, EMBED), jnp.float32)] * 2,
            [pltpu.VMEM((EB, BB, 8, 128), jnp.float32)] * 2,
            pltpu.VMEM((MAXLEN, EMBED), jnp.float32),
            [pltpu.SemaphoreType.DMA] * 2,
            [pltpu.SemaphoreType.DMA] * 2,
        ],
    )(x4, table_flat, pos_table)
    return out5.transpose(2, 4, 0, 1, 3).reshape(BATCH, MAXLEN, EMBED)
